# baseline (speedup 1.0000x reference)
import functools

import jax
import jax.numpy as jnp
from jax import lax
from jax.experimental import pallas as pl
from jax.experimental.pallas import tpu as pltpu
from jax.experimental.pallas import tpu_sc as plsc

D_MODEL = 1024
NUM_ROWS = 16384
NB = 3
CHUNK = 32


@functools.partial(jax.jit, static_argnames=())
def _sc_embed(idx3, table):
    info = plsc.get_sparse_core_info()
    nc, ns = info.num_cores, info.num_subcores
    nw = nc * ns
    per_w = NUM_ROWS // nw
    n_chunks = per_w // CHUNK
    assert idx3.shape == (nw, n_chunks, CHUNK)

    mesh = plsc.VectorSubcoreMesh(core_axis_name="c", subcore_axis_name="s")

    @functools.partial(
        pl.kernel,
        mesh=mesh,
        out_type=jax.ShapeDtypeStruct((NUM_ROWS, D_MODEL), jnp.float32),
        scratch_types=(
            [pltpu.VMEM((n_chunks, CHUNK), jnp.int32)]
            + [pltpu.VMEM((CHUNK, D_MODEL), jnp.float32) for _ in range(NB)]
            + [pltpu.SemaphoreType.DMA for _ in range(2 * NB)]
        ),
    )
    def k(idx_hbm, table_hbm, out_hbm, idx_v, *rest):
        bufs = rest[:NB]
        gsems = rest[NB:2 * NB]
        ssems = rest[2 * NB:]
        wid = lax.axis_index("s") * nc + lax.axis_index("c")
        base = wid * per_w
        pltpu.sync_copy(idx_hbm.at[wid], idx_v)

        g_h = [None] * n_chunks
        s_h = [None] * n_chunks
        for g in range(n_chunks):
            b = g % NB
            if g >= NB:
                s_h[g - NB].wait()
            g_h[g] = pltpu.async_copy(
                table_hbm.at[idx_v.at[g]], bufs[b], gsems[b])
            if g >= 1:
                pb = (g - 1) % NB
                g_h[g - 1].wait()
                s_h[g - 1] = pltpu.async_copy(
                    bufs[pb],
                    out_hbm.at[pl.ds(base + (g - 1) * CHUNK, CHUNK)],
                    ssems[pb])
        lb = (n_chunks - 1) % NB
        g_h[n_chunks - 1].wait()
        s_h[n_chunks - 1] = pltpu.async_copy(
            bufs[lb],
            out_hbm.at[pl.ds(base + (n_chunks - 1) * CHUNK, CHUNK)],
            ssems[lb])
        for g in range(max(0, n_chunks - NB), n_chunks):
            s_h[g].wait()

    return k(idx3, table)


def kernel(segment_input, table):
    info = plsc.get_sparse_core_info()
    nw = info.num_cores * info.num_subcores
    per_w = NUM_ROWS // nw
    n_chunks = per_w // CHUNK
    idx3 = segment_input.astype(jnp.int32).reshape(nw, n_chunks, CHUNK)
    out = _sc_embed(idx3, table)
    return out.reshape(segment_input.shape + (D_MODEL,))


# speedup vs baseline: 5.0437x; 5.0437x over previous
import functools

import jax
import jax.numpy as jnp
from jax import lax
from jax.experimental import pallas as pl
from jax.experimental.pallas import tpu as pltpu
from jax.experimental.pallas import tpu_sc as plsc

D_MODEL = 1024
NUM_ROWS = 16384
NB = 3
CHUNK = 32


@functools.partial(jax.jit, static_argnames=())
def _sc_embed(idx3, table):
    info = plsc.get_sparse_core_info()
    nc, ns = info.num_cores, info.num_subcores
    nw = nc * ns
    per_w = NUM_ROWS // nw
    n_chunks = per_w // CHUNK
    assert idx3.shape == (nw, n_chunks, CHUNK)

    mesh = plsc.VectorSubcoreMesh(core_axis_name="c", subcore_axis_name="s")

    @functools.partial(
        pl.kernel,
        mesh=mesh,
        out_type=jax.ShapeDtypeStruct((NUM_ROWS, D_MODEL), jnp.float32),
        scratch_types=(
            [pltpu.VMEM((n_chunks, CHUNK), jnp.int32)]
            + [pltpu.VMEM((CHUNK, D_MODEL), jnp.float32) for _ in range(NB)]
            + [pltpu.SemaphoreType.DMA for _ in range(2 * NB)]
        ),
    )
    def k(idx_hbm, table_hbm, out_hbm, idx_v, *rest):
        bufs = rest[:NB]
        gsems = rest[NB:2 * NB]
        ssems = rest[2 * NB:]
        wid = lax.axis_index("s") * nc + lax.axis_index("c")
        base = wid * per_w
        pltpu.sync_copy(idx_hbm.at[wid], idx_v)

        g_h = [None] * n_chunks
        s_h = [None] * n_chunks
        for g in range(n_chunks):
            b = g % NB
            if g >= NB:
                s_h[g - NB].wait()
            g_h[g] = pltpu.async_copy(
                table_hbm.at[idx_v.at[g]], bufs[b], gsems[b])
            if g >= 1:
                pb = (g - 1) % NB
                g_h[g - 1].wait()
                s_h[g - 1] = pltpu.async_copy(
                    bufs[pb],
                    out_hbm.at[pl.ds(base + (g - 1) * CHUNK, CHUNK)],
                    ssems[pb])
        lb = (n_chunks - 1) % NB
        g_h[n_chunks - 1].wait()
        s_h[n_chunks - 1] = pltpu.async_copy(
            bufs[lb],
            out_hbm.at[pl.ds(base + (n_chunks - 1) * CHUNK, CHUNK)],
            ssems[lb])
        for g in range(max(0, n_chunks - NB), n_chunks):
            s_h[g].wait()

    return k(idx3, table)


def kernel(segment_input, table):
    info = plsc.get_sparse_core_info()
    nw = info.num_cores * info.num_subcores
    per_w = NUM_ROWS // nw
    n_chunks = per_w // CHUNK
    idx3 = segment_input.astype(jnp.int32).reshape(nw, n_chunks, CHUNK)
    nrows = table.shape[0]
    table_rep = jnp.tile(table, (nw, 1))
    idx3 = idx3 + (jnp.arange(nw, dtype=jnp.int32) * nrows)[:, None, None]
    out = _sc_embed(idx3, table_rep)
    return out.reshape(segment_input.shape + (D_MODEL,))
